# baseline (speedup 1.0000x reference)
import functools

import jax
import jax.numpy as jnp
from jax import lax
from jax.experimental import pallas as pl
from jax.experimental.pallas import tpu as pltpu
from jax.experimental.pallas import tpu_sc as plsc

VOCAB = 49408
NUM_POS = 77
EMBED = 768
BATCH = 1024

L = 16
NC = 2
NS = 16
NW = NC * NS
BATCH_PER_W = BATCH // NW
ROWS_PER_W = BATCH_PER_W * NUM_POS
CHUNK = 8
NCHUNK = ROWS_PER_W // CHUNK
NBUF = 8
DEPTH = NBUF - 2
NOUTER = -(-NCHUNK // NBUF)


def _emb_body(idx_hbm, tok_hbm, pos_hbm, out_hbm,
              idx_v, pos_v,
              b0, b1, b2, b3, b4, b5, b6, b7,
              g0, g1, g2, g3, g4, g5, g6, g7,
              s0, s1, s2, s3, s4, s5, s6, s7):
    bufs = (b0, b1, b2, b3, b4, b5, b6, b7)
    gsem = (g0, g1, g2, g3, g4, g5, g6, g7)
    ssem = (s0, s1, s2, s3, s4, s5, s6, s7)

    wid = lax.axis_index("s") * NC + lax.axis_index("c")
    row0 = wid * ROWS_PER_W

    pltpu.sync_copy(idx_hbm.at[pl.ds(row0, ROWS_PER_W)], idx_v)
    pltpu.sync_copy(pos_hbm, pos_v)

    def gather(c, b):
        return pltpu.make_async_copy(
            tok_hbm.at[idx_v.at[pl.ds(c * CHUNK, CHUNK)]], bufs[b], gsem[b])

    def scatter(c, b):
        return pltpu.make_async_copy(
            bufs[b], out_hbm.at[pl.ds(row0 + c * CHUNK, CHUNK)], ssem[b])

    for b in range(DEPTH):
        gather(b, b).start()

    def outer(t, carry):
        for j in range(NBUF):
            c = t * NBUF + j
            q = c + DEPTH
            bq = (j + DEPTH) % NBUF

            @pl.when(q < NCHUNK)
            def _():
                @pl.when(q >= NBUF)
                def _():
                    scatter(q - NBUF, bq).wait()
                gather(q, bq).start()

            @pl.when(c < NCHUNK)
            def _():
                gather(c, j).wait()

                def row_add(r, carry2):
                    pr = lax.rem(c * CHUNK + r, NUM_POS)
                    for k in range(EMBED // L):
                        sl = pl.ds(k * L, L)
                        plsc.addupdate(bufs[j].at[r, sl], pos_v[pr, sl])
                    return carry2

                lax.fori_loop(0, CHUNK, row_add, 0, unroll=False)

                scatter(c, j).start()
        return carry

    lax.fori_loop(0, NOUTER, outer, 0, unroll=False)

    for j in range(NBUF):
        cc = NCHUNK - NBUF + j
        scatter(cc, cc % NBUF).wait()


@jax.jit
def _emb_call(idx3, token_table, position_table):
    info = plsc.get_sparse_core_info()
    assert info.num_cores == NC and info.num_subcores == NS

    mesh = plsc.VectorSubcoreMesh(core_axis_name="c", subcore_axis_name="s")
    run = functools.partial(
        pl.kernel,
        mesh=mesh,
        out_type=jax.ShapeDtypeStruct((BATCH * NUM_POS, EMBED), jnp.float32),
        scratch_types=(
            [pltpu.VMEM((ROWS_PER_W,), jnp.int32),
             pltpu.VMEM((NUM_POS, EMBED), jnp.float32)]
            + [pltpu.VMEM((CHUNK, EMBED), jnp.float32)] * NBUF
            + [pltpu.SemaphoreType.DMA] * (2 * NBUF)
        ),
    )(_emb_body)
    return run(idx3, token_table, position_table)


def kernel(input_tokens, token_table, position_table):
    idx3 = input_tokens.astype(jnp.int32).reshape(BATCH * NUM_POS)
    out = _emb_call(idx3, token_table.astype(jnp.float32),
                    position_table.astype(jnp.float32))
    return out.reshape(BATCH, NUM_POS, EMBED)


# speedup vs baseline: 2.2149x; 2.2149x over previous
import functools

import jax
import jax.numpy as jnp
from jax import lax
from jax.experimental import pallas as pl
from jax.experimental.pallas import tpu as pltpu
from jax.experimental.pallas import tpu_sc as plsc

VOCAB = 49408
NUM_POS = 77
POS_PAD = 80
EMBED = 768
BATCH = 1024

L = 16
NC = 2
NS = 16
NW = NC * NS
BATCH_PER_W = BATCH // NW
NBUF = 10
CHUNK = 8
DEPTH = NBUF - 2
NCH_W = BATCH_PER_W * NBUF

_CHROWS = [CHUNK] * 9 + [NUM_POS - 9 * CHUNK]


def _emb_body(idx_hbm, tok_hbm, pos_hbm, out_hbm, idx_v, pos_v, *rest):
    bufs = rest[:NBUF]
    gsem = rest[NBUF:2 * NBUF]
    ssem = rest[2 * NBUF:3 * NBUF]

    wid = lax.axis_index("s") * NC + lax.axis_index("c")
    lb0 = wid * BATCH_PER_W

    pltpu.sync_copy(idx_hbm.at[pl.ds(lb0 * POS_PAD, BATCH_PER_W * POS_PAD)],
                    idx_v)
    pltpu.sync_copy(pos_hbm, pos_v)

    def gather(lb, j):
        n = _CHROWS[j]
        src = tok_hbm.at[idx_v.at[pl.ds(lb * POS_PAD + j * CHUNK, n)]]
        return pltpu.make_async_copy(src, bufs[j], gsem[j])

    def scatter(lb, j):
        n = _CHROWS[j]
        dst = out_hbm.at[lb0 + lb, pl.ds(j * CHUNK, n)]
        return pltpu.make_async_copy(bufs[j], dst, ssem[j])

    for j in range(DEPTH):
        gather(0, j).start()

    def outer(lb, carry):
        for j in range(NBUF):
            cglob = lb * NBUF + j
            jq = (j + DEPTH) % NBUF
            lbq = lb + (0 if j + DEPTH < NBUF else 1)

            @pl.when(cglob + DEPTH < NCH_W)
            def _():
                @pl.when(cglob >= 2)
                def _():
                    scatter(lbq - 1, jq).wait()
                gather(lbq, jq).start()

            gather(lb, j).wait()

            def row_add(r, carry2):
                for k in range(EMBED // L):
                    sl = pl.ds(k * L, L)
                    plsc.addupdate(bufs[j].at[r, sl],
                                   pos_v[j * CHUNK + r, sl])
                return carry2

            lax.fori_loop(0, _CHROWS[j], row_add, 0, unroll=False)

            scatter(lb, j).start()
        return carry

    lax.fori_loop(0, BATCH_PER_W, outer, 0, unroll=False)

    for j in range(NBUF):
        scatter(BATCH_PER_W - 1, j).wait()


@jax.jit
def _emb_call(idx_pad, token_table, position_table):
    info = plsc.get_sparse_core_info()
    assert info.num_cores == NC and info.num_subcores == NS

    mesh = plsc.VectorSubcoreMesh(core_axis_name="c", subcore_axis_name="s",
                                  num_cores=NC)
    run = functools.partial(
        pl.kernel,
        mesh=mesh,
        out_type=jax.ShapeDtypeStruct((BATCH, NUM_POS, EMBED), jnp.float32),
        scratch_types=(
            [pltpu.VMEM((BATCH_PER_W * POS_PAD,), jnp.int32),
             pltpu.VMEM((NUM_POS, EMBED), jnp.float32)]
            + [pltpu.VMEM((n, EMBED), jnp.float32) for n in _CHROWS]
            + [pltpu.SemaphoreType.DMA] * (2 * NBUF)
        ),
    )(_emb_body)
    return run(idx_pad, token_table, position_table)


def kernel(input_tokens, token_table, position_table):
    idx = input_tokens.astype(jnp.int32)
    idx_pad = jnp.pad(idx, ((0, 0), (0, POS_PAD - NUM_POS))).reshape(-1)
    return _emb_call(idx_pad, token_table.astype(jnp.float32),
                     position_table.astype(jnp.float32))
